# Initial kernel scaffold; baseline (speedup 1.0000x reference)
import jax
import jax.numpy as jnp
from jax.experimental import pallas as pl


def kernel(edge_index, edge_type, entity_embed, W1, W1_self, W2, W2_self, W_ih, W_hh, b_ih, b_hh):
    raise NotImplementedError("write your pallas kernel here")


# speedup vs baseline: 5.5299x; 5.5299x over previous
import functools

import jax
import jax.numpy as jnp
from jax import lax
from jax.experimental import pallas as pl
from jax.experimental.pallas import tpu as pltpu
from jax.experimental.pallas import tpu_sc as plsc

NC, NS = 2, 16
NW = NC * NS


def _sc_mesh():
    return plsc.VectorSubcoreMesh(core_axis_name="c", subcore_axis_name="s",
                                  num_cores=NC, num_subcores=NS)


def _sc_gather(table, idx):
    M, D = table.shape
    K = idx.shape[0]
    CH = 128
    assert K % CH == 0
    nch = K // CH
    nper = -(-nch // NW)

    @functools.partial(
        pl.kernel, mesh=_sc_mesh(),
        out_type=jax.ShapeDtypeStruct((K, D), jnp.float32),
        scratch_types=[
            pltpu.VMEM((CH,), jnp.int32),
            pltpu.VMEM((CH, D), jnp.float32),
            pltpu.SemaphoreType.DMA,
        ])
    def k(table_hbm, idx_hbm, out_hbm, idx_v, rows_v, sem):
        wid = lax.axis_index("s") * NC + lax.axis_index("c")

        def body(i, carry):
            c = i * NW + wid

            @pl.when(c < nch)
            def _():
                base = c * CH
                pltpu.sync_copy(idx_hbm.at[pl.ds(base, CH)], idx_v)
                pltpu.async_copy(table_hbm.at[idx_v], rows_v, sem).wait()
                pltpu.sync_copy(rows_v, out_hbm.at[pl.ds(base, CH)])
            return carry

        lax.fori_loop(0, nper, body, 0)

    return k(table, idx)


def _sc_scatter(msg, dst, T, E, N, NP):
    D = 128
    NR = 8
    RANGE = NP // NR
    CH = 80
    assert E % CH == 0 and RANGE % 8 == 0
    nch = E // CH
    nper = -(-nch // NS)
    AR = RANGE + 8
    CHZ = max(d for d in range(8, 257, 8) if AR % d == 0)
    nzch = AR // CHZ
    nzper = -(-nzch // NS)
    W15 = -(-(RANGE // NS) // 8) * 8
    WLAST = RANGE - (NS - 1) * W15
    assert W15 % 8 == 0 and WLAST % 8 == 0 and WLAST > 0

    @functools.partial(
        pl.kernel, mesh=_sc_mesh(),
        out_type=jax.ShapeDtypeStruct((T, NP, D), jnp.float32),
        scratch_types=[
            pltpu.VMEM((CH,), jnp.int32),
            pltpu.VMEM((CH, D), jnp.float32),
            pltpu.VMEM((CHZ, D), jnp.float32),
            pltpu.VMEM_SHARED((AR, D), jnp.float32),
        ])
    def k(m_hbm, d_hbm, out_hbm, idx_v, msg_v, z_v, acc):
        KT = T * E
        cid = lax.axis_index("c")
        tid = lax.axis_index("s")

        for zr in range(CHZ):
            for zc in range(D // 16):
                z_v[zr, pl.ds(zc * 16, 16)] = jnp.zeros((16,), jnp.float32)

        for rl in range(NR // 2):
            rr = cid * (NR // 2) + rl
            r0 = rr * RANGE

            def snapshot(t, carry):
                def zbody(i, c2):
                    c = i * NS + tid

                    @pl.when(c < nzch)
                    def _():
                        pltpu.sync_copy(z_v, acc.at[pl.ds(c * CHZ, CHZ)])
                    return c2

                lax.fori_loop(0, nzper, zbody, 0)
                plsc.subcore_barrier()

                def sbody(i, c2):
                    c = i * NS + tid

                    @pl.when(c < nch)
                    def _():
                        base = t * E + c * CH
                        pltpu.sync_copy(d_hbm.at[pl.ds(rr * KT + base, CH)],
                                        idx_v)
                        pltpu.sync_copy(m_hbm.at[pl.ds(base, CH)], msg_v)
                        pltpu.sync_copy(msg_v, acc.at[idx_v], add=True)
                    return c2

                lax.fori_loop(0, nper, sbody, 0)
                plsc.subcore_barrier()

                @pl.when(tid < NS - 1)
                def _():
                    pltpu.sync_copy(
                        acc.at[pl.ds(tid * W15, W15)],
                        out_hbm.at[t, pl.ds(r0 + tid * W15, W15)])

                @pl.when(tid == NS - 1)
                def _():
                    pltpu.sync_copy(
                        acc.at[pl.ds((NS - 1) * W15, WLAST)],
                        out_hbm.at[t, pl.ds(r0 + (NS - 1) * W15, WLAST)])

                plsc.subcore_barrier()
                return carry

            lax.fori_loop(0, T, snapshot, 0)

    return k(msg, dst)


def _blockdiag(W):
    R, NB, DB, _ = W.shape
    eye = jnp.eye(NB, dtype=W.dtype)
    Wd = jnp.einsum('rbio,bc->rbico', W, eye)
    return Wd.reshape(R, NB * DB, NB * DB)


def _tc_relmm(hs, et, Wd):
    K, D = hs.shape
    R = Wd.shape[0]
    BN = 640
    assert K % BN == 0

    def body(et_ref, hs_ref, wd_ref, o_ref):
        h = hs_ref[...]
        e = et_ref[...]
        acc = jnp.zeros((BN, D), jnp.float32)
        for r in range(R):
            sel = jnp.where(e == r, h, 0.0)
            acc = acc + jnp.dot(sel, wd_ref[r],
                                preferred_element_type=jnp.float32)
        o_ref[...] = acc

    return pl.pallas_call(
        body,
        grid=(K // BN,),
        in_specs=[
            pl.BlockSpec((BN, 1), lambda i: (i, 0)),
            pl.BlockSpec((BN, D), lambda i: (i, 0)),
            pl.BlockSpec((R, D, D), lambda i: (0, 0, 0)),
        ],
        out_specs=pl.BlockSpec((BN, D), lambda i: (i, 0)),
        out_shape=jax.ShapeDtypeStruct((K, D), jnp.float32),
    )(et, hs, Wd)


def _tc_h1(agg, e, W_self):
    T, NP, D = agg.shape
    N, _ = e.shape
    BN = 400
    assert N % BN == 0

    def body(a_ref, e_ref, w_ref, o_ref):
        s = jnp.dot(e_ref[...], w_ref[...], preferred_element_type=jnp.float32)
        o_ref[...] = jnp.maximum(a_ref[0] + s, 0.0)[None]

    return pl.pallas_call(
        body,
        grid=(T, N // BN),
        in_specs=[
            pl.BlockSpec((1, BN, D), lambda t, i: (t, i, 0)),
            pl.BlockSpec((BN, D), lambda t, i: (i, 0)),
            pl.BlockSpec((D, D), lambda t, i: (0, 0)),
        ],
        out_specs=pl.BlockSpec((1, BN, D), lambda t, i: (t, i, 0)),
        out_shape=jax.ShapeDtypeStruct((T, N, D), jnp.float32),
    )(agg, e, W_self)


def _tc_maxstage(agg2, h1, W_self):
    T, N, D = h1.shape
    BN = 400

    def body(a_ref, h_ref, w_ref, o_ref):
        m = a_ref[0] + jnp.dot(h_ref[0], w_ref[...],
                               preferred_element_type=jnp.float32)
        red = jnp.max(m, axis=0, keepdims=True)

        @pl.when(pl.program_id(1) == 0)
        def _():
            o_ref[0] = red

        @pl.when(pl.program_id(1) > 0)
        def _():
            o_ref[0] = jnp.maximum(o_ref[0], red)

    return pl.pallas_call(
        body,
        grid=(T, N // BN),
        in_specs=[
            pl.BlockSpec((1, BN, D), lambda t, i: (t, i, 0)),
            pl.BlockSpec((1, BN, D), lambda t, i: (t, i, 0)),
            pl.BlockSpec((D, D), lambda t, i: (0, 0)),
        ],
        out_specs=pl.BlockSpec((1, 1, D), lambda t, i: (t, 0, 0)),
        out_shape=jax.ShapeDtypeStruct((T, 1, D), jnp.float32),
    )(agg2, h1, W_self).reshape(T, D)


def _tc_gru(gh, WihT, WhhT, bi, bh):
    T, D = gh.shape

    def sigmoid(x):
        return 1.0 / (1.0 + jnp.exp(-x))

    def body(gh_ref, wi_ref, wh_ref, bi_ref, bh_ref, o_ref):
        h = jnp.zeros((1, D), jnp.float32)
        for t in range(T):
            x = gh_ref[t:t + 1, :]
            gi = jnp.dot(x, wi_ref[...],
                         preferred_element_type=jnp.float32) + bi_ref[...]
            gg = jnp.dot(h, wh_ref[...],
                         preferred_element_type=jnp.float32) + bh_ref[...]
            i_r, i_z, i_n = gi[:, :D], gi[:, D:2 * D], gi[:, 2 * D:]
            h_r, h_z, h_n = gg[:, :D], gg[:, D:2 * D], gg[:, 2 * D:]
            r = sigmoid(i_r + h_r)
            z = sigmoid(i_z + h_z)
            n = jnp.tanh(i_n + r * h_n)
            h = (1.0 - z) * n + z * h
            o_ref[t:t + 1, :] = h

    return pl.pallas_call(
        body,
        out_shape=jax.ShapeDtypeStruct((T, D), jnp.float32),
    )(gh, WihT, WhhT, bi, bh)


def kernel(edge_index, edge_type, entity_embed, W1, W1_self, W2, W2_self,
           W_ih, W_hh, b_ih, b_hh):
    T, _, E = edge_index.shape
    N, D = entity_embed.shape
    K = T * E

    ei = edge_index.astype(jnp.int32)
    src = ei[:, 0, :].reshape(K)
    dst = ei[:, 1, :].reshape(K)
    et = edge_type.astype(jnp.int32).reshape(K, 1)

    Wd1 = _blockdiag(W1)
    Wd2 = _blockdiag(W2)

    NP = -(-N // 64) * 64

    NR = 8
    RANGE = NP // NR
    rid = jnp.arange(NR, dtype=jnp.int32)[:, None]
    loc = dst[None, :] - rid * RANGE
    dst_rr = jnp.where((loc >= 0) & (loc < RANGE),
                       loc, RANGE + (dst[None, :] & 7)).reshape(-1)

    hs1 = _sc_gather(entity_embed, src)
    msg1 = _tc_relmm(hs1, et, Wd1)
    agg1 = _sc_scatter(msg1, dst_rr, T, E, N, NP)
    h1 = _tc_h1(agg1, entity_embed, W1_self)

    gidx2 = (src.reshape(T, E)
             + (jnp.arange(T, dtype=jnp.int32) * N)[:, None]).reshape(K)
    hs2 = _sc_gather(h1.reshape(T * N, D), gidx2)
    msg2 = _tc_relmm(hs2, et, Wd2)
    agg2 = _sc_scatter(msg2, dst_rr, T, E, N, NP)

    gh = _tc_maxstage(agg2, h1, W2_self)
    outs = _tc_gru(gh, W_ih.T, W_hh.T, b_ih.reshape(1, -1), b_hh.reshape(1, -1))
    return outs, outs[-1]


# speedup vs baseline: 7.0724x; 1.2789x over previous
import functools

import jax
import jax.numpy as jnp
from jax import lax
from jax.experimental import pallas as pl
from jax.experimental.pallas import tpu as pltpu
from jax.experimental.pallas import tpu_sc as plsc

NC, NS = 2, 16
NW = NC * NS


def _sc_mesh():
    return plsc.VectorSubcoreMesh(core_axis_name="c", subcore_axis_name="s",
                                  num_cores=NC, num_subcores=NS)


def _sc_gather(table, idx):
    M, D = table.shape
    K = idx.shape[0]
    CH = 128
    assert K % CH == 0
    nch = K // CH
    nper = -(-nch // NW)

    @functools.partial(
        pl.kernel, mesh=_sc_mesh(),
        out_type=jax.ShapeDtypeStruct((K, D), jnp.float32),
        scratch_types=[
            pltpu.VMEM((CH,), jnp.int32),
            pltpu.VMEM((CH, D), jnp.float32),
            pltpu.SemaphoreType.DMA,
        ])
    def k(table_hbm, idx_hbm, out_hbm, idx_v, rows_v, sem):
        wid = lax.axis_index("s") * NC + lax.axis_index("c")

        def body(i, carry):
            c = i * NW + wid

            @pl.when(c < nch)
            def _():
                base = c * CH
                pltpu.sync_copy(idx_hbm.at[pl.ds(base, CH)], idx_v)
                pltpu.async_copy(table_hbm.at[idx_v], rows_v, sem).wait()
                pltpu.sync_copy(rows_v, out_hbm.at[pl.ds(base, CH)])
            return carry

        lax.fori_loop(0, nper, body, 0)

    return k(table, idx)


def _sc_scatter(msg, dst, T, E, N, NP):
    D = 128
    NR = 8
    RANGE = NP // NR
    CH = 80
    assert E % CH == 0 and RANGE % 8 == 0
    nch = E // CH
    NBCH = 2
    BE = NBCH * CH
    assert E % BE == 0
    nb = E // BE
    nper = -(-nb // NS)
    AR = RANGE + 8
    CHZ = max(d for d in range(8, 129, 8) if AR % d == 0)
    nzch = AR // CHZ
    nzper = -(-nzch // NS)
    W15 = -(-(RANGE // NS) // 8) * 8
    WLAST = RANGE - (NS - 1) * W15
    assert W15 % 8 == 0 and WLAST % 8 == 0 and WLAST > 0

    @functools.partial(
        pl.kernel, mesh=_sc_mesh(),
        out_type=jax.ShapeDtypeStruct((T, NP, D), jnp.float32),
        scratch_types=[
            pltpu.VMEM((NBCH, CH), jnp.int32),
            pltpu.VMEM((NBCH * CH, D), jnp.float32),
            pltpu.VMEM((CHZ, D), jnp.float32),
            pltpu.VMEM_SHARED((AR, D), jnp.float32),
            pltpu.SemaphoreType.DMA,
        ])
    def k(m_hbm, d_hbm, out_hbm, idx_v, msg_v, z_v, acc, sem):
        KT = T * E
        cid = lax.axis_index("c")
        tid = lax.axis_index("s")

        for zr in range(CHZ):
            for zc in range(D // 16):
                z_v[zr, pl.ds(zc * 16, 16)] = jnp.zeros((16,), jnp.float32)

        for rl in range(NR // 2):
            rr = cid * (NR // 2) + rl
            r0 = rr * RANGE

            def snapshot(t, carry):
                def zbody(i, c2):
                    c = i * NS + tid

                    @pl.when(c < nzch)
                    def _():
                        pltpu.sync_copy(z_v, acc.at[pl.ds(c * CHZ, CHZ)])
                    return c2

                lax.fori_loop(0, nzper, zbody, 0)
                plsc.subcore_barrier()

                def sbody(i, c2):
                    b = i * NS + tid

                    @pl.when(b < nb)
                    def _():
                        base = t * E + b * BE
                        descs = [
                            pltpu.async_copy(
                                d_hbm.at[pl.ds(rr * KT + base + j * CH, CH)],
                                idx_v.at[j], sem)
                            for j in range(NBCH)
                        ]
                        descs.append(pltpu.async_copy(
                            m_hbm.at[pl.ds(base, BE)], msg_v, sem))
                        for dsc in descs:
                            dsc.wait()
                        for j in range(NBCH):
                            pltpu.sync_copy(msg_v.at[pl.ds(j * CH, CH)],
                                            acc.at[idx_v.at[j]], add=True)
                    return c2

                lax.fori_loop(0, nper, sbody, 0)
                plsc.subcore_barrier()

                @pl.when(tid < NS - 1)
                def _():
                    pltpu.sync_copy(
                        acc.at[pl.ds(tid * W15, W15)],
                        out_hbm.at[t, pl.ds(r0 + tid * W15, W15)])

                @pl.when(tid == NS - 1)
                def _():
                    pltpu.sync_copy(
                        acc.at[pl.ds((NS - 1) * W15, WLAST)],
                        out_hbm.at[t, pl.ds(r0 + (NS - 1) * W15, WLAST)])

                plsc.subcore_barrier()
                return carry

            lax.fori_loop(0, T, snapshot, 0)

    return k(msg, dst)


def _blockdiag(W):
    R, NB, DB, _ = W.shape
    eye = jnp.eye(NB, dtype=W.dtype)
    Wd = jnp.einsum('rbio,bc->rbico', W, eye)
    return Wd.reshape(R, NB * DB, NB * DB)


def _tc_relmm(hs, et, Wd):
    K, D = hs.shape
    R = Wd.shape[0]
    BN = 640
    assert K % BN == 0

    def body(et_ref, hs_ref, wd_ref, o_ref):
        h = hs_ref[...]
        e = et_ref[...]
        acc = jnp.zeros((BN, D), jnp.float32)
        for r in range(R):
            sel = jnp.where(e == r, h, 0.0)
            acc = acc + jnp.dot(sel, wd_ref[r],
                                preferred_element_type=jnp.float32)
        o_ref[...] = acc

    return pl.pallas_call(
        body,
        grid=(K // BN,),
        in_specs=[
            pl.BlockSpec((BN, 1), lambda i: (i, 0)),
            pl.BlockSpec((BN, D), lambda i: (i, 0)),
            pl.BlockSpec((R, D, D), lambda i: (0, 0, 0)),
        ],
        out_specs=pl.BlockSpec((BN, D), lambda i: (i, 0)),
        out_shape=jax.ShapeDtypeStruct((K, D), jnp.float32),
    )(et, hs, Wd)


def _tc_h1(agg, e, W_self):
    T, NP, D = agg.shape
    N, _ = e.shape
    BN = 400
    assert N % BN == 0

    def body(a_ref, e_ref, w_ref, o_ref):
        s = jnp.dot(e_ref[...], w_ref[...], preferred_element_type=jnp.float32)
        o_ref[...] = jnp.maximum(a_ref[0] + s, 0.0)[None]

    return pl.pallas_call(
        body,
        grid=(T, N // BN),
        in_specs=[
            pl.BlockSpec((1, BN, D), lambda t, i: (t, i, 0)),
            pl.BlockSpec((BN, D), lambda t, i: (i, 0)),
            pl.BlockSpec((D, D), lambda t, i: (0, 0)),
        ],
        out_specs=pl.BlockSpec((1, BN, D), lambda t, i: (t, i, 0)),
        out_shape=jax.ShapeDtypeStruct((T, N, D), jnp.float32),
    )(agg, e, W_self)


def _tc_maxstage(agg2, h1, W_self):
    T, N, D = h1.shape
    BN = 400

    def body(a_ref, h_ref, w_ref, o_ref):
        m = a_ref[0] + jnp.dot(h_ref[0], w_ref[...],
                               preferred_element_type=jnp.float32)
        red = jnp.max(m, axis=0, keepdims=True)

        @pl.when(pl.program_id(1) == 0)
        def _():
            o_ref[0] = red

        @pl.when(pl.program_id(1) > 0)
        def _():
            o_ref[0] = jnp.maximum(o_ref[0], red)

    return pl.pallas_call(
        body,
        grid=(T, N // BN),
        in_specs=[
            pl.BlockSpec((1, BN, D), lambda t, i: (t, i, 0)),
            pl.BlockSpec((1, BN, D), lambda t, i: (t, i, 0)),
            pl.BlockSpec((D, D), lambda t, i: (0, 0)),
        ],
        out_specs=pl.BlockSpec((1, 1, D), lambda t, i: (t, 0, 0)),
        out_shape=jax.ShapeDtypeStruct((T, 1, D), jnp.float32),
    )(agg2, h1, W_self).reshape(T, D)


def _tc_gru(gh, WihT, WhhT, bi, bh):
    T, D = gh.shape

    def sigmoid(x):
        return 1.0 / (1.0 + jnp.exp(-x))

    def body(gh_ref, wi_ref, wh_ref, bi_ref, bh_ref, o_ref):
        h = jnp.zeros((1, D), jnp.float32)
        for t in range(T):
            x = gh_ref[t:t + 1, :]
            gi = jnp.dot(x, wi_ref[...],
                         preferred_element_type=jnp.float32) + bi_ref[...]
            gg = jnp.dot(h, wh_ref[...],
                         preferred_element_type=jnp.float32) + bh_ref[...]
            i_r, i_z, i_n = gi[:, :D], gi[:, D:2 * D], gi[:, 2 * D:]
            h_r, h_z, h_n = gg[:, :D], gg[:, D:2 * D], gg[:, 2 * D:]
            r = sigmoid(i_r + h_r)
            z = sigmoid(i_z + h_z)
            n = jnp.tanh(i_n + r * h_n)
            h = (1.0 - z) * n + z * h
            o_ref[t:t + 1, :] = h

    return pl.pallas_call(
        body,
        out_shape=jax.ShapeDtypeStruct((T, D), jnp.float32),
    )(gh, WihT, WhhT, bi, bh)


def kernel(edge_index, edge_type, entity_embed, W1, W1_self, W2, W2_self,
           W_ih, W_hh, b_ih, b_hh):
    T, _, E = edge_index.shape
    N, D = entity_embed.shape
    K = T * E

    ei = edge_index.astype(jnp.int32)
    src = ei[:, 0, :].reshape(K)
    dst = ei[:, 1, :].reshape(K)
    et = edge_type.astype(jnp.int32).reshape(K, 1)

    Wd1 = _blockdiag(W1)
    Wd2 = _blockdiag(W2)

    NP = -(-N // 64) * 64

    NR = 8
    RANGE = NP // NR
    rid = jnp.arange(NR, dtype=jnp.int32)[:, None]
    loc = dst[None, :] - rid * RANGE
    dst_rr = jnp.where((loc >= 0) & (loc < RANGE),
                       loc, RANGE + (dst[None, :] & 7)).reshape(-1)

    hs1 = _sc_gather(entity_embed, src)
    msg1 = _tc_relmm(hs1, et, Wd1)
    agg1 = _sc_scatter(msg1, dst_rr, T, E, N, NP)
    h1 = _tc_h1(agg1, entity_embed, W1_self)

    gidx2 = (src.reshape(T, E)
             + (jnp.arange(T, dtype=jnp.int32) * N)[:, None]).reshape(K)
    hs2 = _sc_gather(h1.reshape(T * N, D), gidx2)
    msg2 = _tc_relmm(hs2, et, Wd2)
    agg2 = _sc_scatter(msg2, dst_rr, T, E, N, NP)

    gh = _tc_maxstage(agg2, h1, W2_self)
    outs = _tc_gru(gh, W_ih.T, W_hh.T, b_ih.reshape(1, -1), b_hh.reshape(1, -1))
    return outs, outs[-1]


# speedup vs baseline: 7.2224x; 1.0212x over previous
import functools

import jax
import jax.numpy as jnp
from jax import lax
from jax.experimental import pallas as pl
from jax.experimental.pallas import tpu as pltpu
from jax.experimental.pallas import tpu_sc as plsc

NC, NS = 2, 16
NW = NC * NS


def _sc_mesh():
    return plsc.VectorSubcoreMesh(core_axis_name="c", subcore_axis_name="s",
                                  num_cores=NC, num_subcores=NS)


def _sc_gather(table, idx):
    M, D = table.shape
    K = idx.shape[0]
    CH = 128
    NB = 5
    assert K % (CH * NB) == 0
    nbat = K // (CH * NB)
    nper = -(-nbat // NW)

    @functools.partial(
        pl.kernel, mesh=_sc_mesh(),
        out_type=jax.ShapeDtypeStruct((K, D), jnp.float32),
        scratch_types=[
            pltpu.VMEM((NB, CH), jnp.int32),
            pltpu.VMEM((NB * CH, D), jnp.float32),
            pltpu.SemaphoreType.DMA,
        ])
    def k(table_hbm, idx_hbm, out_hbm, idx_v, rows_v, sem):
        wid = lax.axis_index("s") * NC + lax.axis_index("c")

        def body(i, carry):
            b = i * NW + wid

            @pl.when(b < nbat)
            def _():
                base = b * NB * CH
                ds1 = [pltpu.async_copy(
                    idx_hbm.at[pl.ds(base + j * CH, CH)], idx_v.at[j], sem)
                    for j in range(NB)]
                for dsc in ds1:
                    dsc.wait()
                ds2 = [pltpu.async_copy(
                    table_hbm.at[idx_v.at[j]],
                    rows_v.at[pl.ds(j * CH, CH)], sem)
                    for j in range(NB)]
                for dsc in ds2:
                    dsc.wait()
                ds3 = [pltpu.async_copy(
                    rows_v.at[pl.ds(j * CH, CH)],
                    out_hbm.at[pl.ds(base + j * CH, CH)], sem)
                    for j in range(NB)]
                for dsc in ds3:
                    dsc.wait()
            return carry

        lax.fori_loop(0, nper, body, 0)

    return k(table, idx)


def _sc_scatter(msg, dst, T, E, N, NP):
    D = 128
    NR = 8
    RANGE = NP // NR
    CH = 80
    assert E % CH == 0 and RANGE % 8 == 0
    nch = E // CH
    NBCH = 2
    BE = NBCH * CH
    assert E % BE == 0
    nb = E // BE
    nper = -(-nb // NS)
    AR = RANGE + 8
    CHZ = max(d for d in range(8, 129, 8) if AR % d == 0)
    nzch = AR // CHZ
    nzper = -(-nzch // NS)
    W15 = -(-(RANGE // NS) // 8) * 8
    WLAST = RANGE - (NS - 1) * W15
    assert W15 % 8 == 0 and WLAST % 8 == 0 and WLAST > 0

    @functools.partial(
        pl.kernel, mesh=_sc_mesh(),
        out_type=jax.ShapeDtypeStruct((T, NP, D), jnp.float32),
        scratch_types=[
            pltpu.VMEM((NBCH, CH), jnp.int32),
            pltpu.VMEM((NBCH * CH, D), jnp.float32),
            pltpu.VMEM((CHZ, D), jnp.float32),
            pltpu.VMEM_SHARED((AR, D), jnp.float32),
            pltpu.SemaphoreType.DMA,
        ])
    def k(m_hbm, d_hbm, out_hbm, idx_v, msg_v, z_v, acc, sem):
        KT = T * E
        cid = lax.axis_index("c")
        tid = lax.axis_index("s")

        for zr in range(CHZ):
            for zc in range(D // 16):
                z_v[zr, pl.ds(zc * 16, 16)] = jnp.zeros((16,), jnp.float32)

        for rl in range(NR // 2):
            rr = cid * (NR // 2) + rl
            r0 = rr * RANGE

            def snapshot(t, carry):
                def zbody(i, c2):
                    c = i * NS + tid

                    @pl.when(c < nzch)
                    def _():
                        pltpu.sync_copy(z_v, acc.at[pl.ds(c * CHZ, CHZ)])
                    return c2

                lax.fori_loop(0, nzper, zbody, 0)
                plsc.subcore_barrier()

                def sbody(i, c2):
                    b = i * NS + tid

                    @pl.when(b < nb)
                    def _():
                        base = t * E + b * BE
                        descs = [
                            pltpu.async_copy(
                                d_hbm.at[pl.ds(rr * KT + base + j * CH, CH)],
                                idx_v.at[j], sem)
                            for j in range(NBCH)
                        ]
                        descs.append(pltpu.async_copy(
                            m_hbm.at[pl.ds(base, BE)], msg_v, sem))
                        for dsc in descs:
                            dsc.wait()
                        for j in range(NBCH):
                            pltpu.sync_copy(msg_v.at[pl.ds(j * CH, CH)],
                                            acc.at[idx_v.at[j]], add=True)
                    return c2

                lax.fori_loop(0, nper, sbody, 0)
                plsc.subcore_barrier()

                @pl.when(tid < NS - 1)
                def _():
                    pltpu.sync_copy(
                        acc.at[pl.ds(tid * W15, W15)],
                        out_hbm.at[t, pl.ds(r0 + tid * W15, W15)])

                @pl.when(tid == NS - 1)
                def _():
                    pltpu.sync_copy(
                        acc.at[pl.ds((NS - 1) * W15, WLAST)],
                        out_hbm.at[t, pl.ds(r0 + (NS - 1) * W15, WLAST)])

                plsc.subcore_barrier()
                return carry

            lax.fori_loop(0, T, snapshot, 0)

    return k(msg, dst)


def _blockdiag(W):
    R, NB, DB, _ = W.shape
    eye = jnp.eye(NB, dtype=W.dtype)
    Wd = jnp.einsum('rbio,bc->rbico', W, eye)
    return Wd.reshape(R, NB * DB, NB * DB)


def _tc_relmm(hs, et, Wd):
    K, D = hs.shape
    R = Wd.shape[0]
    BN = 640
    assert K % BN == 0

    def body(et_ref, hs_ref, wd_ref, o_ref):
        h = hs_ref[...]
        e = et_ref[...]
        acc = jnp.zeros((BN, D), jnp.float32)
        for r in range(R):
            sel = jnp.where(e == r, h, 0.0)
            acc = acc + jnp.dot(sel, wd_ref[r],
                                preferred_element_type=jnp.float32)
        o_ref[...] = acc

    return pl.pallas_call(
        body,
        grid=(K // BN,),
        in_specs=[
            pl.BlockSpec((BN, 1), lambda i: (i, 0)),
            pl.BlockSpec((BN, D), lambda i: (i, 0)),
            pl.BlockSpec((R, D, D), lambda i: (0, 0, 0)),
        ],
        out_specs=pl.BlockSpec((BN, D), lambda i: (i, 0)),
        out_shape=jax.ShapeDtypeStruct((K, D), jnp.float32),
    )(et, hs, Wd)


def _tc_h1(agg, e, W_self):
    T, NP, D = agg.shape
    N, _ = e.shape
    BN = 400
    assert N % BN == 0

    def body(a_ref, e_ref, w_ref, o_ref):
        s = jnp.dot(e_ref[...], w_ref[...], preferred_element_type=jnp.float32)
        o_ref[...] = jnp.maximum(a_ref[0] + s, 0.0)[None]

    return pl.pallas_call(
        body,
        grid=(T, N // BN),
        in_specs=[
            pl.BlockSpec((1, BN, D), lambda t, i: (t, i, 0)),
            pl.BlockSpec((BN, D), lambda t, i: (i, 0)),
            pl.BlockSpec((D, D), lambda t, i: (0, 0)),
        ],
        out_specs=pl.BlockSpec((1, BN, D), lambda t, i: (t, i, 0)),
        out_shape=jax.ShapeDtypeStruct((T, N, D), jnp.float32),
    )(agg, e, W_self)


def _tc_maxstage(agg2, h1, W_self):
    T, N, D = h1.shape
    BN = 400

    def body(a_ref, h_ref, w_ref, o_ref):
        m = a_ref[0] + jnp.dot(h_ref[0], w_ref[...],
                               preferred_element_type=jnp.float32)
        red = jnp.max(m, axis=0, keepdims=True)

        @pl.when(pl.program_id(1) == 0)
        def _():
            o_ref[0] = red

        @pl.when(pl.program_id(1) > 0)
        def _():
            o_ref[0] = jnp.maximum(o_ref[0], red)

    return pl.pallas_call(
        body,
        grid=(T, N // BN),
        in_specs=[
            pl.BlockSpec((1, BN, D), lambda t, i: (t, i, 0)),
            pl.BlockSpec((1, BN, D), lambda t, i: (t, i, 0)),
            pl.BlockSpec((D, D), lambda t, i: (0, 0)),
        ],
        out_specs=pl.BlockSpec((1, 1, D), lambda t, i: (t, 0, 0)),
        out_shape=jax.ShapeDtypeStruct((T, 1, D), jnp.float32),
    )(agg2, h1, W_self).reshape(T, D)


def _tc_gru(gh, WihT, WhhT, bi, bh):
    T, D = gh.shape

    def sigmoid(x):
        return 1.0 / (1.0 + jnp.exp(-x))

    def body(gh_ref, wi_ref, wh_ref, bi_ref, bh_ref, o_ref):
        h = jnp.zeros((1, D), jnp.float32)
        for t in range(T):
            x = gh_ref[t:t + 1, :]
            gi = jnp.dot(x, wi_ref[...],
                         preferred_element_type=jnp.float32) + bi_ref[...]
            gg = jnp.dot(h, wh_ref[...],
                         preferred_element_type=jnp.float32) + bh_ref[...]
            i_r, i_z, i_n = gi[:, :D], gi[:, D:2 * D], gi[:, 2 * D:]
            h_r, h_z, h_n = gg[:, :D], gg[:, D:2 * D], gg[:, 2 * D:]
            r = sigmoid(i_r + h_r)
            z = sigmoid(i_z + h_z)
            n = jnp.tanh(i_n + r * h_n)
            h = (1.0 - z) * n + z * h
            o_ref[t:t + 1, :] = h

    return pl.pallas_call(
        body,
        out_shape=jax.ShapeDtypeStruct((T, D), jnp.float32),
    )(gh, WihT, WhhT, bi, bh)


def kernel(edge_index, edge_type, entity_embed, W1, W1_self, W2, W2_self,
           W_ih, W_hh, b_ih, b_hh):
    T, _, E = edge_index.shape
    N, D = entity_embed.shape
    K = T * E

    ei = edge_index.astype(jnp.int32)
    src = ei[:, 0, :].reshape(K)
    dst = ei[:, 1, :].reshape(K)
    et = edge_type.astype(jnp.int32).reshape(K, 1)

    Wd1 = _blockdiag(W1)
    Wd2 = _blockdiag(W2)

    NP = -(-N // 64) * 64

    NR = 8
    RANGE = NP // NR
    rid = jnp.arange(NR, dtype=jnp.int32)[:, None]
    loc = dst[None, :] - rid * RANGE
    dst_rr = jnp.where((loc >= 0) & (loc < RANGE),
                       loc, RANGE + (dst[None, :] & 7)).reshape(-1)

    hs1 = _sc_gather(entity_embed, src)
    msg1 = _tc_relmm(hs1, et, Wd1)
    agg1 = _sc_scatter(msg1, dst_rr, T, E, N, NP)
    h1 = _tc_h1(agg1, entity_embed, W1_self)

    gidx2 = (src.reshape(T, E)
             + (jnp.arange(T, dtype=jnp.int32) * N)[:, None]).reshape(K)
    hs2 = _sc_gather(h1.reshape(T * N, D), gidx2)
    msg2 = _tc_relmm(hs2, et, Wd2)
    agg2 = _sc_scatter(msg2, dst_rr, T, E, N, NP)

    gh = _tc_maxstage(agg2, h1, W2_self)
    outs = _tc_gru(gh, W_ih.T, W_hh.T, b_ih.reshape(1, -1), b_hh.reshape(1, -1))
    return outs, outs[-1]
